# baseline (speedup 1.0000x reference)
import functools

import jax
import jax.numpy as jnp
from jax import lax
from jax.experimental import pallas as pl
from jax.experimental.pallas import tpu as pltpu
from jax.experimental.pallas import tpu_sc as plsc

H = 128
N_NODES = 100000
NU_ = 50000
NB_ = 50000
BLK = 1000

_NC = 2
_NS = 16
_NW = _NC * _NS
_CH = 128


def _embed_body(x_ref, w_ref, b_ref, o_ref):
    x = x_ref[...].astype(jnp.bfloat16)
    o_ref[...] = jnp.tanh(
        jnp.dot(x, w_ref[...], preferred_element_type=jnp.float32) + b_ref[...]
    ).astype(jnp.bfloat16)


def _embed(x, w, b, blk):
    n = x.shape[0]
    return pl.pallas_call(
        _embed_body,
        grid=(n // blk,),
        in_specs=[
            pl.BlockSpec((blk, H), lambda i: (i, 0)),
            pl.BlockSpec((H, H), lambda i: (0, 0)),
            pl.BlockSpec((1, H), lambda i: (0, 0)),
        ],
        out_specs=pl.BlockSpec((blk, H), lambda i: (i, 0)),
        out_shape=jax.ShapeDtypeStruct((n, H), jnp.bfloat16),
    )(x, w, b)


def _pad_idx(idx_flat, steps):
    total = _NW * steps * _CH
    return jnp.concatenate(
        [idx_flat, jnp.zeros((total - idx_flat.shape[0],), jnp.int32)]
    ).reshape(_NW, steps, _CH)


def _sc_gather(table, idx3d):
    steps = idx3d.shape[1]
    total = _NW * steps * _CH
    mesh = plsc.VectorSubcoreMesh(core_axis_name="c", subcore_axis_name="s")

    @functools.partial(
        pl.kernel,
        mesh=mesh,
        out_type=jax.ShapeDtypeStruct((total, H), jnp.float32),
        scratch_types=[
            pltpu.VMEM((steps, _CH), jnp.int32),
            pltpu.VMEM((_CH, H), jnp.float32),
            pltpu.SemaphoreType.DMA,
        ],
    )
    def gather_kernel(table_hbm, idx_hbm, out_hbm, idx_v, rows_v, sem):
        wid = lax.axis_index("s") * _NC + lax.axis_index("c")
        row0 = wid * steps
        pltpu.sync_copy(idx_hbm.at[wid], idx_v)

        def body(j, carry):
            pltpu.async_copy(table_hbm.at[idx_v.at[j]], rows_v, sem).wait()
            pltpu.sync_copy(rows_v, out_hbm.at[pl.ds((row0 + j) * _CH, _CH)])
            return carry

        lax.fori_loop(0, steps, body, 0)

    return gather_kernel(table, idx3d)


def _dot(a, w):
    return jnp.dot(a, w, preferred_element_type=jnp.float32)


def _t(v):
    return v.astype(jnp.bfloat16)


def _msg_and_node(r0, emb, msg_layers, node_ws, node_bs):
    (w1, b1) = msg_layers[0]
    r = jnp.tanh(_dot(r0, w1[...]) + b1[...])
    for (w_ref, b_ref) in msg_layers[1:]:
        r = jnp.tanh(
            _dot(_t(r), w_ref[0:H]) + _dot(r0, w_ref[H : 2 * H]) + b_ref[...]
        )
    r = _t(r)
    wn0 = node_ws[0]
    e = jnp.tanh(_dot(emb, wn0[0:H]) + _dot(r, wn0[H : 2 * H]) + node_bs[0][...])
    for (w_ref, b_ref) in zip(node_ws[1:], node_bs[1:]):
        e = jnp.tanh(
            _dot(_t(e), w_ref[0:H]) + _dot(emb, w_ref[H : 2 * H]) + b_ref[...]
        )
    return e


def _chain_u_body(
    xu_ref, emb_ref, we_ref, be_ref,
    w0, w1, w2, w3, w4, w5, b0, b1, b2, b3, b4, b5,
    wn0, wn1, wn2, wn3, wn4, bn0, bn1, bn2, bn3, bn4,
    o_ref,
):
    emb = emb_ref[...]
    m = _t(jnp.tanh(_dot(_t(xu_ref[...]), we_ref[...]) + be_ref[...]))
    r0 = _t(jnp.tanh(_dot(m, w0[...]) + b0[...]))
    o_ref[...] = _msg_and_node(
        r0, emb,
        ((w1, b1), (w2, b2), (w3, b3), (w4, b4), (w5, b5)),
        (wn0, wn1, wn2, wn3, wn4), (bn0, bn1, bn2, bn3, bn4),
    )


def _chain_b_body(
    xb1_ref, xb2_ref, emb_ref, we_ref, be_ref,
    w0, w1, w2, w3, w4, w5, b0, b1, b2, b3, b4, b5,
    wn0, wn1, wn2, wn3, wn4, bn0, bn1, bn2, bn3, bn4,
    carry_ref, o_ref,
):
    emb = emb_ref[...]
    m1 = _t(jnp.tanh(_dot(_t(xb1_ref[...]), we_ref[...]) + be_ref[...]))
    m2 = _t(jnp.tanh(_dot(_t(xb2_ref[...]), we_ref[...]) + be_ref[...]))
    s0 = _t(jnp.tanh(_dot(m1, w0[0:H]) + _dot(m2, w0[H : 2 * H]) + b0[...]))
    o_ref[...] = _msg_and_node(
        s0, emb,
        ((w1, b1), (w2, b2), (w3, b3), (w4, b4), (w5, b5)),
        (wn0, wn1, wn2, wn3, wn4), (bn0, bn1, bn2, bn3, bn4),
    )


_W = pl.BlockSpec((H, H), lambda i: (0, 0))
_W2 = pl.BlockSpec((2 * H, H), lambda i: (0, 0))
_B = pl.BlockSpec((1, H), lambda i: (0, 0))


def _chain_u(gu, emb, we, be, ws, bs, wns, bns):
    return pl.pallas_call(
        _chain_u_body,
        grid=(NU_ // BLK,),
        in_specs=[
            pl.BlockSpec((BLK, H), lambda i: (i, 0)),
            pl.BlockSpec((BLK, H), lambda i: (i, 0)),
            _W, _B,
            _W, _W, _W2, _W2, _W2, _W2,
            _B, _B, _B, _B, _B, _B,
            _W2, _W2, _W2, _W2, _W2,
            _B, _B, _B, _B, _B,
        ],
        out_specs=pl.BlockSpec((BLK, H), lambda i: (i, 0)),
        out_shape=jax.ShapeDtypeStruct((N_NODES, H), jnp.float32),
    )(gu, emb, we, be, *ws, *bs, *wns, *bns)


def _chain_b(gb, emb, we, be, ws, bs, wns, bns, carry):
    nu_b = NU_ // BLK
    return pl.pallas_call(
        _chain_b_body,
        grid=(NB_ // BLK,),
        in_specs=[
            pl.BlockSpec((BLK, H), lambda i: (i, 0)),
            pl.BlockSpec((BLK, H), lambda i: (i + nu_b, 0)),
            pl.BlockSpec((BLK, H), lambda i: (i + nu_b, 0)),
            _W, _B,
            _W2, _W, _W2, _W2, _W2, _W2,
            _B, _B, _B, _B, _B, _B,
            _W2, _W2, _W2, _W2, _W2,
            _B, _B, _B, _B, _B,
            pl.BlockSpec(memory_space=pl.ANY),
        ],
        out_specs=pl.BlockSpec((BLK, H), lambda i: (i + nu_b, 0)),
        out_shape=jax.ShapeDtypeStruct((N_NODES, H), jnp.float32),
        input_output_aliases={27: 0},
    )(gb, gb, emb, we, be, *ws, *bs, *wns, *bns, carry)


def kernel(node_feats, unary_src, binary_src, params):
    p = params
    bf16 = jnp.bfloat16

    idx_u = _pad_idx(unary_src, 13)
    idx_b = _pad_idx(
        jnp.concatenate([binary_src[:, 0], binary_src[:, 1]]), 25
    )
    we = p["We"].astype(bf16)
    be = p["be"].reshape(1, H)
    we, idx_u, idx_b = lax.optimization_barrier((we, idx_u, idx_b))
    gu = _sc_gather(node_feats, idx_u)
    gb = _sc_gather(node_feats, idx_b)
    emb = _embed(node_feats, we, be, 2000)

    def wc(n):
        return p["W" + n].astype(bf16)

    def b2d(n):
        return p["b" + n].reshape(1, H)

    wsu = [wc("u%d" % i) for i in range(6)]
    bsu = [b2d("u%d" % i) for i in range(6)]
    wsb = [wc("b%d" % i) for i in range(6)]
    bsb = [b2d("b%d" % i) for i in range(6)]
    wns = [wc("n%d" % i) for i in range(5)]
    bns = [b2d("n%d" % i) for i in range(5)]

    e_u = _chain_u(gu, emb, we, be, wsu, bsu, wns, bns)
    return _chain_b(gb, emb, we, be, wsb, bsb, wns, bns, e_u)


# speedup vs baseline: 1.1174x; 1.1174x over previous
import functools

import jax
import jax.numpy as jnp
from jax import lax
from jax.experimental import pallas as pl
from jax.experimental.pallas import tpu as pltpu
from jax.experimental.pallas import tpu_sc as plsc

H = 128
N_NODES = 100000
NU_ = 50000
NB_ = 50000
BLK = 1000

_NC = 2
_NS = 16
_NW = _NC * _NS
_CH = 128


def _embed_body(x_ref, w_ref, b_ref, o_ref):
    x = x_ref[...].astype(jnp.bfloat16)
    o_ref[...] = jnp.tanh(
        jnp.dot(x, w_ref[...], preferred_element_type=jnp.float32) + b_ref[...]
    ).astype(jnp.bfloat16)


def _embed(x, w, b, blk):
    n = x.shape[0]
    return pl.pallas_call(
        _embed_body,
        grid=(n // blk,),
        in_specs=[
            pl.BlockSpec((blk, H), lambda i: (i, 0)),
            pl.BlockSpec((H, H), lambda i: (0, 0)),
            pl.BlockSpec((1, H), lambda i: (0, 0)),
        ],
        out_specs=pl.BlockSpec((blk, H), lambda i: (i, 0)),
        out_shape=jax.ShapeDtypeStruct((n, H), jnp.bfloat16),
    )(x, w, b)


def _pad_idx(idx_flat, steps):
    total = _NW * steps * _CH
    return jnp.concatenate(
        [idx_flat, jnp.zeros((total - idx_flat.shape[0],), jnp.int32)]
    ).reshape(_NW, steps, _CH)


def _sc_gather(table, idx3d):
    steps = idx3d.shape[1]
    total = _NW * steps * _CH
    mesh = plsc.VectorSubcoreMesh(core_axis_name="c", subcore_axis_name="s")

    @functools.partial(
        pl.kernel,
        mesh=mesh,
        out_type=jax.ShapeDtypeStruct((total, H), jnp.float32),
        scratch_types=[
            pltpu.VMEM((steps, _CH), jnp.int32),
            pltpu.VMEM((_CH, H), jnp.float32),
            pltpu.SemaphoreType.DMA,
        ],
    )
    def gather_kernel(table_hbm, idx_hbm, out_hbm, idx_v, rows_v, sem):
        wid = lax.axis_index("s") * _NC + lax.axis_index("c")
        row0 = wid * steps
        pltpu.sync_copy(idx_hbm.at[wid], idx_v)

        def body(j, carry):
            pltpu.async_copy(table_hbm.at[idx_v.at[j]], rows_v, sem).wait()
            pltpu.sync_copy(rows_v, out_hbm.at[pl.ds((row0 + j) * _CH, _CH)])
            return carry

        lax.fori_loop(0, steps, body, 0)

    return gather_kernel(table, idx3d)


def _dot(a, w):
    return jnp.dot(a, w, preferred_element_type=jnp.float32)


def _t(v):
    return v.astype(jnp.bfloat16)


def _msg_and_node(r0, emb, msg_layers, node_ws, node_bs):
    (w1, b1) = msg_layers[0]
    r = jnp.tanh(_dot(r0, w1[...]) + b1[...])
    for (w_ref, b_ref) in msg_layers[1:]:
        r = jnp.tanh(
            _dot(jnp.concatenate([_t(r), r0], axis=1), w_ref[...]) + b_ref[...]
        )
    r = _t(r)
    e = jnp.tanh(
        _dot(jnp.concatenate([emb, r], axis=1), node_ws[0][...]) + node_bs[0][...]
    )
    for (w_ref, b_ref) in zip(node_ws[1:], node_bs[1:]):
        e = jnp.tanh(
            _dot(jnp.concatenate([_t(e), emb], axis=1), w_ref[...]) + b_ref[...]
        )
    return e


def _chain_u_body(
    xu_ref, emb_ref, we_ref, be_ref,
    w0, w1, w2, w3, w4, w5, b0, b1, b2, b3, b4, b5,
    wn0, wn1, wn2, wn3, wn4, bn0, bn1, bn2, bn3, bn4,
    o_ref,
):
    emb = emb_ref[...]
    m = _t(jnp.tanh(_dot(_t(xu_ref[...]), we_ref[...]) + be_ref[...]))
    r0 = _t(jnp.tanh(_dot(m, w0[...]) + b0[...]))
    o_ref[...] = _msg_and_node(
        r0, emb,
        ((w1, b1), (w2, b2), (w3, b3), (w4, b4), (w5, b5)),
        (wn0, wn1, wn2, wn3, wn4), (bn0, bn1, bn2, bn3, bn4),
    )


def _chain_b_body(
    xb1_ref, xb2_ref, emb_ref, we_ref, be_ref,
    w0, w1, w2, w3, w4, w5, b0, b1, b2, b3, b4, b5,
    wn0, wn1, wn2, wn3, wn4, bn0, bn1, bn2, bn3, bn4,
    carry_ref, o_ref,
):
    emb = emb_ref[...]
    m1 = _t(jnp.tanh(_dot(_t(xb1_ref[...]), we_ref[...]) + be_ref[...]))
    m2 = _t(jnp.tanh(_dot(_t(xb2_ref[...]), we_ref[...]) + be_ref[...]))
    s0 = _t(
        jnp.tanh(_dot(jnp.concatenate([m1, m2], axis=1), w0[...]) + b0[...])
    )
    o_ref[...] = _msg_and_node(
        s0, emb,
        ((w1, b1), (w2, b2), (w3, b3), (w4, b4), (w5, b5)),
        (wn0, wn1, wn2, wn3, wn4), (bn0, bn1, bn2, bn3, bn4),
    )


_W = pl.BlockSpec((H, H), lambda i: (0, 0))
_W2 = pl.BlockSpec((2 * H, H), lambda i: (0, 0))
_B = pl.BlockSpec((1, H), lambda i: (0, 0))


def _chain_u(gu, emb, we, be, ws, bs, wns, bns):
    return pl.pallas_call(
        _chain_u_body,
        grid=(NU_ // BLK,),
        in_specs=[
            pl.BlockSpec((BLK, H), lambda i: (i, 0)),
            pl.BlockSpec((BLK, H), lambda i: (i, 0)),
            _W, _B,
            _W, _W, _W2, _W2, _W2, _W2,
            _B, _B, _B, _B, _B, _B,
            _W2, _W2, _W2, _W2, _W2,
            _B, _B, _B, _B, _B,
        ],
        out_specs=pl.BlockSpec((BLK, H), lambda i: (i, 0)),
        out_shape=jax.ShapeDtypeStruct((N_NODES, H), jnp.float32),
    )(gu, emb, we, be, *ws, *bs, *wns, *bns)


def _chain_b(gb, emb, we, be, ws, bs, wns, bns, carry):
    nu_b = NU_ // BLK
    return pl.pallas_call(
        _chain_b_body,
        grid=(NB_ // BLK,),
        in_specs=[
            pl.BlockSpec((BLK, H), lambda i: (i, 0)),
            pl.BlockSpec((BLK, H), lambda i: (i + nu_b, 0)),
            pl.BlockSpec((BLK, H), lambda i: (i + nu_b, 0)),
            _W, _B,
            _W2, _W, _W2, _W2, _W2, _W2,
            _B, _B, _B, _B, _B, _B,
            _W2, _W2, _W2, _W2, _W2,
            _B, _B, _B, _B, _B,
            pl.BlockSpec(memory_space=pl.ANY),
        ],
        out_specs=pl.BlockSpec((BLK, H), lambda i: (i + nu_b, 0)),
        out_shape=jax.ShapeDtypeStruct((N_NODES, H), jnp.float32),
        input_output_aliases={27: 0},
    )(gb, gb, emb, we, be, *ws, *bs, *wns, *bns, carry)


def kernel(node_feats, unary_src, binary_src, params):
    p = params
    bf16 = jnp.bfloat16

    idx_u = _pad_idx(unary_src, 13)
    idx_b = _pad_idx(
        jnp.concatenate([binary_src[:, 0], binary_src[:, 1]]), 25
    )
    we = p["We"].astype(bf16)
    be = p["be"].reshape(1, H)
    we, idx_u, idx_b = lax.optimization_barrier((we, idx_u, idx_b))
    gu = _sc_gather(node_feats, idx_u)
    gb = _sc_gather(node_feats, idx_b)
    emb = _embed(node_feats, we, be, 2000)

    def wc(n):
        return p["W" + n].astype(bf16)

    def b2d(n):
        return p["b" + n].reshape(1, H)

    wsu = [wc("u%d" % i) for i in range(6)]
    bsu = [b2d("u%d" % i) for i in range(6)]
    wsb = [wc("b%d" % i) for i in range(6)]
    bsb = [b2d("b%d" % i) for i in range(6)]
    wns = [wc("n%d" % i) for i in range(5)]
    bns = [b2d("n%d" % i) for i in range(5)]

    e_u = _chain_u(gu, emb, we, be, wsu, bsu, wns, bns)
    return _chain_b(gb, emb, we, be, wsb, bsb, wns, bns, e_u)


# speedup vs baseline: 1.2928x; 1.1570x over previous
import functools

import jax
import jax.numpy as jnp
from jax import lax
from jax.experimental import pallas as pl
from jax.experimental.pallas import tpu as pltpu
from jax.experimental.pallas import tpu_sc as plsc

H = 128
N_NODES = 100000
NU_ = 50000
NB_ = 50000
BLK = 1000

_NC = 2
_NS = 16
_NW = _NC * _NS
_CH = 128


def _embed_body(x_ref, w_ref, b_ref, o_ref):
    x = x_ref[...].astype(jnp.bfloat16)
    o_ref[...] = jnp.tanh(
        jnp.dot(x, w_ref[...], preferred_element_type=jnp.float32) + b_ref[...]
    ).astype(jnp.bfloat16)


def _embed(x, w, b, blk):
    n = x.shape[0]
    return pl.pallas_call(
        _embed_body,
        grid=(n // blk,),
        in_specs=[
            pl.BlockSpec((blk, H), lambda i: (i, 0)),
            pl.BlockSpec((H, H), lambda i: (0, 0)),
            pl.BlockSpec((1, H), lambda i: (0, 0)),
        ],
        out_specs=pl.BlockSpec((blk, H), lambda i: (i, 0)),
        out_shape=jax.ShapeDtypeStruct((n, H), jnp.bfloat16),
    )(x, w, b)


def _pad_idx(idx_flat, steps0, steps1):
    sp = steps0 + steps1
    total = 16 * sp * _CH
    flat = jnp.concatenate(
        [idx_flat, jnp.zeros((total - idx_flat.shape[0],), jnp.int32)]
    ).reshape(16, sp * _CH)
    maxs = max(steps0, steps1)
    i0 = flat[:, : steps0 * _CH].reshape(16, steps0, _CH)
    i1 = flat[:, steps0 * _CH :].reshape(16, steps1, _CH)
    z0 = jnp.zeros((16, maxs - steps0, _CH), jnp.int32)
    z1 = jnp.zeros((16, maxs - steps1, _CH), jnp.int32)
    i0 = jnp.concatenate([i0, z0], axis=1)
    i1 = jnp.concatenate([i1, z1], axis=1)
    return jnp.stack([i0, i1], axis=1).reshape(_NW, maxs, _CH)


def _sc_gather(table, idx3d, steps0, steps1):
    maxs = idx3d.shape[1]
    total = 16 * (steps0 + steps1) * _CH
    mesh = plsc.VectorSubcoreMesh(core_axis_name="c", subcore_axis_name="s")

    @functools.partial(
        pl.kernel,
        mesh=mesh,
        out_type=jax.ShapeDtypeStruct((total, H), jnp.float32),
        scratch_types=[
            pltpu.VMEM((maxs, _CH), jnp.int32),
            pltpu.VMEM((_CH, H), jnp.float32),
            pltpu.SemaphoreType.DMA,
        ],
    )
    def gather_kernel(table_hbm, idx_hbm, out_hbm, idx_v, rows_v, sem):
        s = lax.axis_index("s")
        c = lax.axis_index("c")
        wid = s * _NC + c
        my_steps = steps0 + c * (steps1 - steps0)
        row0 = s * (steps0 + steps1) + c * steps0
        pltpu.sync_copy(idx_hbm.at[wid], idx_v)

        def body(j, carry):
            pltpu.async_copy(table_hbm.at[idx_v.at[j]], rows_v, sem).wait()
            pltpu.sync_copy(rows_v, out_hbm.at[pl.ds((row0 + j) * _CH, _CH)])
            return carry

        lax.fori_loop(0, my_steps, body, 0)

    return gather_kernel(table, idx3d)


def _dot(a, w):
    return jnp.dot(a, w, preferred_element_type=jnp.float32)


def _t(v):
    return v.astype(jnp.bfloat16)


def _msg_and_node(r0, emb, msg_layers, node_ws, node_bs):
    (w1, b1) = msg_layers[0]
    r = jnp.tanh(_dot(r0, w1[...]) + b1[...])
    for (w_ref, b_ref) in msg_layers[1:]:
        r = jnp.tanh(
            _dot(jnp.concatenate([_t(r), r0], axis=1), w_ref[...]) + b_ref[...]
        )
    r = _t(r)
    e = jnp.tanh(
        _dot(jnp.concatenate([emb, r], axis=1), node_ws[0][...]) + node_bs[0][...]
    )
    for (w_ref, b_ref) in zip(node_ws[1:], node_bs[1:]):
        e = jnp.tanh(
            _dot(jnp.concatenate([_t(e), emb], axis=1), w_ref[...]) + b_ref[...]
        )
    return e


def _chain_u_body(
    xu_ref, emb_ref, we_ref, be_ref,
    w0, w1, w2, w3, w4, w5, b0, b1, b2, b3, b4, b5,
    wn0, wn1, wn2, wn3, wn4, bn0, bn1, bn2, bn3, bn4,
    o_ref,
):
    emb = emb_ref[...]
    m = _t(jnp.tanh(_dot(_t(xu_ref[...]), we_ref[...]) + be_ref[...]))
    r0 = _t(jnp.tanh(_dot(m, w0[...]) + b0[...]))
    o_ref[...] = _msg_and_node(
        r0, emb,
        ((w1, b1), (w2, b2), (w3, b3), (w4, b4), (w5, b5)),
        (wn0, wn1, wn2, wn3, wn4), (bn0, bn1, bn2, bn3, bn4),
    )


def _chain_b_body(
    xb1_ref, xb2_ref, emb_ref, we_ref, be_ref,
    w0, w1, w2, w3, w4, w5, b0, b1, b2, b3, b4, b5,
    wn0, wn1, wn2, wn3, wn4, bn0, bn1, bn2, bn3, bn4,
    carry_ref, o_ref,
):
    emb = emb_ref[...]
    m1 = _t(jnp.tanh(_dot(_t(xb1_ref[...]), we_ref[...]) + be_ref[...]))
    m2 = _t(jnp.tanh(_dot(_t(xb2_ref[...]), we_ref[...]) + be_ref[...]))
    s0 = _t(
        jnp.tanh(_dot(jnp.concatenate([m1, m2], axis=1), w0[...]) + b0[...])
    )
    o_ref[...] = _msg_and_node(
        s0, emb,
        ((w1, b1), (w2, b2), (w3, b3), (w4, b4), (w5, b5)),
        (wn0, wn1, wn2, wn3, wn4), (bn0, bn1, bn2, bn3, bn4),
    )


_W = pl.BlockSpec((H, H), lambda i: (0, 0))
_W2 = pl.BlockSpec((2 * H, H), lambda i: (0, 0))
_B = pl.BlockSpec((1, H), lambda i: (0, 0))


def _chain_u(gu, emb, we, be, ws, bs, wns, bns):
    return pl.pallas_call(
        _chain_u_body,
        grid=(NU_ // BLK,),
        in_specs=[
            pl.BlockSpec((BLK, H), lambda i: (i, 0)),
            pl.BlockSpec((BLK, H), lambda i: (i, 0)),
            _W, _B,
            _W, _W, _W2, _W2, _W2, _W2,
            _B, _B, _B, _B, _B, _B,
            _W2, _W2, _W2, _W2, _W2,
            _B, _B, _B, _B, _B,
        ],
        out_specs=pl.BlockSpec((BLK, H), lambda i: (i, 0)),
        out_shape=jax.ShapeDtypeStruct((N_NODES, H), jnp.float32),
    )(gu, emb, we, be, *ws, *bs, *wns, *bns)


def _chain_b(gb, emb, we, be, ws, bs, wns, bns, carry):
    nu_b = NU_ // BLK
    return pl.pallas_call(
        _chain_b_body,
        grid=(NB_ // BLK,),
        in_specs=[
            pl.BlockSpec((BLK, H), lambda i: (i, 0)),
            pl.BlockSpec((BLK, H), lambda i: (i + nu_b, 0)),
            pl.BlockSpec((BLK, H), lambda i: (i + nu_b, 0)),
            _W, _B,
            _W2, _W, _W2, _W2, _W2, _W2,
            _B, _B, _B, _B, _B, _B,
            _W2, _W2, _W2, _W2, _W2,
            _B, _B, _B, _B, _B,
            pl.BlockSpec(memory_space=pl.ANY),
        ],
        out_specs=pl.BlockSpec((BLK, H), lambda i: (i + nu_b, 0)),
        out_shape=jax.ShapeDtypeStruct((N_NODES, H), jnp.float32),
        input_output_aliases={27: 0},
    )(gb, gb, emb, we, be, *ws, *bs, *wns, *bns, carry)


def kernel(node_feats, unary_src, binary_src, params):
    p = params
    bf16 = jnp.bfloat16

    u0, u1 = 17, 9
    b0_, b1_ = 33, 16
    idx_u = _pad_idx(unary_src, u0, u1)
    idx_b = _pad_idx(
        jnp.concatenate([binary_src[:, 0], binary_src[:, 1]]), b0_, b1_
    )
    we = p["We"].astype(bf16)
    be = p["be"].reshape(1, H)
    we, idx_u, idx_b = lax.optimization_barrier((we, idx_u, idx_b))
    gu = _sc_gather(node_feats, idx_u, u0, u1)
    gb = _sc_gather(node_feats, idx_b, b0_, b1_)
    emb = _embed(node_feats, we, be, 2000)

    def wc(n):
        return p["W" + n].astype(bf16)

    def b2d(n):
        return p["b" + n].reshape(1, H)

    wsu = [wc("u%d" % i) for i in range(6)]
    bsu = [b2d("u%d" % i) for i in range(6)]
    wsb = [wc("b%d" % i) for i in range(6)]
    bsb = [b2d("b%d" % i) for i in range(6)]
    wns = [wc("n%d" % i) for i in range(5)]
    bns = [b2d("n%d" % i) for i in range(5)]

    e_u = _chain_u(gu, emb, we, be, wsu, bsu, wns, bns)
    return _chain_b(gb, emb, we, be, wsb, bsb, wns, bns, e_u)


# speedup vs baseline: 1.3506x; 1.0448x over previous
import functools

import jax
import jax.numpy as jnp
from jax import lax
from jax.experimental import pallas as pl
from jax.experimental.pallas import tpu as pltpu
from jax.experimental.pallas import tpu_sc as plsc

H = 128
N_NODES = 100000
NU_ = 50000
NB_ = 50000
BLK = 1000

_NC = 2
_NS = 16
_NW = _NC * _NS
_CH = 128


def _embed_body(x_ref, w_ref, b_ref, o_ref):
    x = x_ref[...].astype(jnp.bfloat16)
    o_ref[...] = jnp.tanh(
        jnp.dot(x, w_ref[...], preferred_element_type=jnp.float32) + b_ref[...]
    ).astype(jnp.bfloat16)


def _embed(x, w, b, blk):
    n = x.shape[0]
    return pl.pallas_call(
        _embed_body,
        grid=(n // blk,),
        in_specs=[
            pl.BlockSpec((blk, H), lambda i: (i, 0)),
            pl.BlockSpec((H, H), lambda i: (0, 0)),
            pl.BlockSpec((1, H), lambda i: (0, 0)),
        ],
        out_specs=pl.BlockSpec((blk, H), lambda i: (i, 0)),
        out_shape=jax.ShapeDtypeStruct((n, H), jnp.bfloat16),
    )(x, w, b)


def _pad_idx(idx_flat, steps0, steps1):
    sp = steps0 + steps1
    total = 16 * sp * _CH
    flat = jnp.concatenate(
        [idx_flat, jnp.zeros((total - idx_flat.shape[0],), jnp.int32)]
    ).reshape(16, sp * _CH)
    maxs = max(steps0, steps1)
    i0 = flat[:, : steps0 * _CH].reshape(16, steps0, _CH)
    i1 = flat[:, steps0 * _CH :].reshape(16, steps1, _CH)
    z0 = jnp.zeros((16, maxs - steps0, _CH), jnp.int32)
    z1 = jnp.zeros((16, maxs - steps1, _CH), jnp.int32)
    i0 = jnp.concatenate([i0, z0], axis=1)
    i1 = jnp.concatenate([i1, z1], axis=1)
    return jnp.stack([i0, i1], axis=1).reshape(_NW, maxs, _CH)


def _sc_gather(table, idx3d, steps0, steps1):
    maxs = idx3d.shape[1]
    total = 16 * (steps0 + steps1) * _CH
    mesh = plsc.VectorSubcoreMesh(core_axis_name="c", subcore_axis_name="s")

    @functools.partial(
        pl.kernel,
        mesh=mesh,
        out_type=jax.ShapeDtypeStruct((total, H), jnp.float32),
        scratch_types=[
            pltpu.VMEM((maxs, _CH), jnp.int32),
            pltpu.VMEM((_CH, H), jnp.float32),
            pltpu.SemaphoreType.DMA,
        ],
    )
    def gather_kernel(table_hbm, idx_hbm, out_hbm, idx_v, rows_v, sem):
        s = lax.axis_index("s")
        c = lax.axis_index("c")
        wid = s * _NC + c
        my_steps = steps0 + c * (steps1 - steps0)
        row0 = s * (steps0 + steps1) + c * steps0
        pltpu.sync_copy(idx_hbm.at[wid], idx_v)

        def body(j, carry):
            pltpu.async_copy(table_hbm.at[idx_v.at[j]], rows_v, sem).wait()
            pltpu.sync_copy(rows_v, out_hbm.at[pl.ds((row0 + j) * _CH, _CH)])
            return carry

        lax.fori_loop(0, my_steps, body, 0)

    return gather_kernel(table, idx3d)


def _dot(a, w):
    return jnp.dot(a, w, preferred_element_type=jnp.float32)


def _t(v):
    return v.astype(jnp.bfloat16)


def _msg_and_node(r0, emb, msg_layers, node_ws, node_bs):
    (w1, b1) = msg_layers[0]
    r = jnp.tanh(_dot(r0, w1[...]) + b1[...])
    for (w_ref, b_ref) in msg_layers[1:]:
        r = jnp.tanh(
            _dot(jnp.concatenate([_t(r), r0], axis=1), w_ref[...]) + b_ref[...]
        )
    r = _t(r)
    e = jnp.tanh(
        _dot(jnp.concatenate([emb, r], axis=1), node_ws[0][...]) + node_bs[0][...]
    )
    for (w_ref, b_ref) in zip(node_ws[1:], node_bs[1:]):
        e = jnp.tanh(
            _dot(jnp.concatenate([_t(e), emb], axis=1), w_ref[...]) + b_ref[...]
        )
    return e


def _chain_u_body(
    xu_ref, emb_ref, we_ref, be_ref,
    w0, w1, w2, w3, w4, w5, b0, b1, b2, b3, b4, b5,
    wn0, wn1, wn2, wn3, wn4, bn0, bn1, bn2, bn3, bn4,
    o_ref,
):
    emb = emb_ref[...]
    m = _t(jnp.tanh(_dot(_t(xu_ref[...]), we_ref[...]) + be_ref[...]))
    r0 = _t(jnp.tanh(_dot(m, w0[...]) + b0[...]))
    o_ref[...] = _msg_and_node(
        r0, emb,
        ((w1, b1), (w2, b2), (w3, b3), (w4, b4), (w5, b5)),
        (wn0, wn1, wn2, wn3, wn4), (bn0, bn1, bn2, bn3, bn4),
    )


def _chain_b_body(
    xb1_ref, xb2_ref, emb_ref, we_ref, be_ref,
    w0, w1, w2, w3, w4, w5, b0, b1, b2, b3, b4, b5,
    wn0, wn1, wn2, wn3, wn4, bn0, bn1, bn2, bn3, bn4,
    carry_ref, o_ref,
):
    emb = emb_ref[...]
    m1 = _t(jnp.tanh(_dot(_t(xb1_ref[...]), we_ref[...]) + be_ref[...]))
    m2 = _t(jnp.tanh(_dot(_t(xb2_ref[...]), we_ref[...]) + be_ref[...]))
    s0 = _t(
        jnp.tanh(_dot(jnp.concatenate([m1, m2], axis=1), w0[...]) + b0[...])
    )
    o_ref[...] = _msg_and_node(
        s0, emb,
        ((w1, b1), (w2, b2), (w3, b3), (w4, b4), (w5, b5)),
        (wn0, wn1, wn2, wn3, wn4), (bn0, bn1, bn2, bn3, bn4),
    )


_W = pl.BlockSpec((H, H), lambda i: (0, 0))
_W2 = pl.BlockSpec((2 * H, H), lambda i: (0, 0))
_B = pl.BlockSpec((1, H), lambda i: (0, 0))


def _chain_u(gu, emb, we, be, ws, bs, wns, bns):
    return pl.pallas_call(
        _chain_u_body,
        grid=(NU_ // BLK,),
        in_specs=[
            pl.BlockSpec((BLK, H), lambda i: (i, 0)),
            pl.BlockSpec((BLK, H), lambda i: (i, 0)),
            _W, _B,
            _W, _W, _W2, _W2, _W2, _W2,
            _B, _B, _B, _B, _B, _B,
            _W2, _W2, _W2, _W2, _W2,
            _B, _B, _B, _B, _B,
        ],
        out_specs=pl.BlockSpec((BLK, H), lambda i: (i, 0)),
        out_shape=jax.ShapeDtypeStruct((N_NODES, H), jnp.float32),
    )(gu, emb, we, be, *ws, *bs, *wns, *bns)


def _chain_b(gb, emb, we, be, ws, bs, wns, bns, carry):
    nu_b = NU_ // BLK
    return pl.pallas_call(
        _chain_b_body,
        grid=(NB_ // BLK,),
        in_specs=[
            pl.BlockSpec((BLK, H), lambda i: (i, 0)),
            pl.BlockSpec((BLK, H), lambda i: (i + nu_b, 0)),
            pl.BlockSpec((BLK, H), lambda i: (i + nu_b, 0)),
            _W, _B,
            _W2, _W, _W2, _W2, _W2, _W2,
            _B, _B, _B, _B, _B, _B,
            _W2, _W2, _W2, _W2, _W2,
            _B, _B, _B, _B, _B,
            pl.BlockSpec(memory_space=pl.ANY),
        ],
        out_specs=pl.BlockSpec((BLK, H), lambda i: (i + nu_b, 0)),
        out_shape=jax.ShapeDtypeStruct((N_NODES, H), jnp.float32),
        input_output_aliases={27: 0},
    )(gb, gb, emb, we, be, *ws, *bs, *wns, *bns, carry)


def kernel(node_feats, unary_src, binary_src, params):
    p = params
    bf16 = jnp.bfloat16

    u0, u1 = 15, 11
    b0_, b1_ = 33, 16
    idx_u = _pad_idx(unary_src, u0, u1)
    idx_b = _pad_idx(
        jnp.concatenate([binary_src[:, 0], binary_src[:, 1]]), b0_, b1_
    )
    we = p["We"].astype(bf16)
    be = p["be"].reshape(1, H)
    we, idx_u, idx_b = lax.optimization_barrier((we, idx_u, idx_b))
    gu = _sc_gather(node_feats, idx_u, u0, u1)
    gb = _sc_gather(node_feats, idx_b, b0_, b1_)
    emb = _embed(node_feats, we, be, 2000)

    def wc(n):
        return p["W" + n].astype(bf16)

    def b2d(n):
        return p["b" + n].reshape(1, H)

    wsu = [wc("u%d" % i) for i in range(6)]
    bsu = [b2d("u%d" % i) for i in range(6)]
    wsb = [wc("b%d" % i) for i in range(6)]
    bsb = [b2d("b%d" % i) for i in range(6)]
    wns = [wc("n%d" % i) for i in range(5)]
    bns = [b2d("n%d" % i) for i in range(5)]

    e_u = _chain_u(gu, emb, we, be, wsu, bsu, wns, bns)
    return _chain_b(gb, emb, we, be, wsb, bsb, wns, bns, e_u)


# speedup vs baseline: 1.4274x; 1.0568x over previous
import functools

import jax
import jax.numpy as jnp
from jax import lax
from jax.experimental import pallas as pl
from jax.experimental.pallas import tpu as pltpu
from jax.experimental.pallas import tpu_sc as plsc

H = 128
N_NODES = 100000
NU_ = 50000
NB_ = 50000
BLK = 2000

_NC = 2
_NS = 16
_NW = _NC * _NS
_CH = 128


def _embed_body(x_ref, w_ref, b_ref, o_ref):
    x = x_ref[...].astype(jnp.bfloat16)
    o_ref[...] = jnp.tanh(
        jnp.dot(x, w_ref[...], preferred_element_type=jnp.float32) + b_ref[...]
    ).astype(jnp.bfloat16)


def _embed(x, w, b, blk):
    n = x.shape[0]
    return pl.pallas_call(
        _embed_body,
        grid=(n // blk,),
        in_specs=[
            pl.BlockSpec((blk, H), lambda i: (i, 0)),
            pl.BlockSpec((H, H), lambda i: (0, 0)),
            pl.BlockSpec((1, H), lambda i: (0, 0)),
        ],
        out_specs=pl.BlockSpec((blk, H), lambda i: (i, 0)),
        out_shape=jax.ShapeDtypeStruct((n, H), jnp.bfloat16),
    )(x, w, b)


def _pad_idx(idx_flat, steps0, steps1):
    sp = steps0 + steps1
    total = 16 * sp * _CH
    flat = jnp.concatenate(
        [idx_flat, jnp.zeros((total - idx_flat.shape[0],), jnp.int32)]
    ).reshape(16, sp * _CH)
    maxs = max(steps0, steps1)
    i0 = flat[:, : steps0 * _CH].reshape(16, steps0, _CH)
    i1 = flat[:, steps0 * _CH :].reshape(16, steps1, _CH)
    z0 = jnp.zeros((16, maxs - steps0, _CH), jnp.int32)
    z1 = jnp.zeros((16, maxs - steps1, _CH), jnp.int32)
    i0 = jnp.concatenate([i0, z0], axis=1)
    i1 = jnp.concatenate([i1, z1], axis=1)
    return jnp.stack([i0, i1], axis=1).reshape(_NW, maxs, _CH)


def _sc_gather(table, idx3d, steps0, steps1):
    maxs = idx3d.shape[1]
    total = 16 * (steps0 + steps1) * _CH
    mesh = plsc.VectorSubcoreMesh(core_axis_name="c", subcore_axis_name="s")

    @functools.partial(
        pl.kernel,
        mesh=mesh,
        out_type=jax.ShapeDtypeStruct((total, H), jnp.float32),
        scratch_types=[
            pltpu.VMEM((maxs, _CH), jnp.int32),
            pltpu.VMEM((_CH, H), jnp.float32),
            pltpu.SemaphoreType.DMA,
        ],
    )
    def gather_kernel(table_hbm, idx_hbm, out_hbm, idx_v, rows_v, sem):
        s = lax.axis_index("s")
        c = lax.axis_index("c")
        wid = s * _NC + c
        my_steps = steps0 + c * (steps1 - steps0)
        row0 = s * (steps0 + steps1) + c * steps0
        pltpu.sync_copy(idx_hbm.at[wid], idx_v)

        def body(j, carry):
            pltpu.async_copy(table_hbm.at[idx_v.at[j]], rows_v, sem).wait()
            pltpu.sync_copy(rows_v, out_hbm.at[pl.ds((row0 + j) * _CH, _CH)])
            return carry

        lax.fori_loop(0, my_steps, body, 0)

    return gather_kernel(table, idx3d)


def _dot(a, w):
    return jnp.dot(a, w, preferred_element_type=jnp.float32)


def _t(v):
    return v.astype(jnp.bfloat16)


def _msg_and_node(r0, emb, msg_layers, node_ws, node_bs):
    (w1, b1) = msg_layers[0]
    r = jnp.tanh(_dot(r0, w1[...]) + b1[...])
    for (w_ref, b_ref) in msg_layers[1:]:
        r = jnp.tanh(
            _dot(jnp.concatenate([_t(r), r0], axis=1), w_ref[...]) + b_ref[...]
        )
    r = _t(r)
    e = jnp.tanh(
        _dot(jnp.concatenate([emb, r], axis=1), node_ws[0][...]) + node_bs[0][...]
    )
    for (w_ref, b_ref) in zip(node_ws[1:], node_bs[1:]):
        e = jnp.tanh(
            _dot(jnp.concatenate([_t(e), emb], axis=1), w_ref[...]) + b_ref[...]
        )
    return e


def _chain_u_body(
    xu_ref, emb_ref, we_ref, be_ref,
    w0, w1, w2, w3, w4, w5, b0, b1, b2, b3, b4, b5,
    wn0, wn1, wn2, wn3, wn4, bn0, bn1, bn2, bn3, bn4,
    o_ref,
):
    emb = emb_ref[...]
    m = _t(jnp.tanh(_dot(_t(xu_ref[...]), we_ref[...]) + be_ref[...]))
    r0 = _t(jnp.tanh(_dot(m, w0[...]) + b0[...]))
    o_ref[...] = _msg_and_node(
        r0, emb,
        ((w1, b1), (w2, b2), (w3, b3), (w4, b4), (w5, b5)),
        (wn0, wn1, wn2, wn3, wn4), (bn0, bn1, bn2, bn3, bn4),
    )


def _chain_b_body(
    xb1_ref, xb2_ref, emb_ref, we_ref, be_ref,
    w0, w1, w2, w3, w4, w5, b0, b1, b2, b3, b4, b5,
    wn0, wn1, wn2, wn3, wn4, bn0, bn1, bn2, bn3, bn4,
    carry_ref, o_ref,
):
    emb = emb_ref[...]
    m1 = _t(jnp.tanh(_dot(_t(xb1_ref[...]), we_ref[...]) + be_ref[...]))
    m2 = _t(jnp.tanh(_dot(_t(xb2_ref[...]), we_ref[...]) + be_ref[...]))
    s0 = _t(
        jnp.tanh(_dot(jnp.concatenate([m1, m2], axis=1), w0[...]) + b0[...])
    )
    o_ref[...] = _msg_and_node(
        s0, emb,
        ((w1, b1), (w2, b2), (w3, b3), (w4, b4), (w5, b5)),
        (wn0, wn1, wn2, wn3, wn4), (bn0, bn1, bn2, bn3, bn4),
    )


_W = pl.BlockSpec((H, H), lambda i: (0, 0))
_W2 = pl.BlockSpec((2 * H, H), lambda i: (0, 0))
_B = pl.BlockSpec((1, H), lambda i: (0, 0))


def _chain_u(gu, emb, we, be, ws, bs, wns, bns):
    return pl.pallas_call(
        _chain_u_body,
        grid=(NU_ // BLK,),
        in_specs=[
            pl.BlockSpec((BLK, H), lambda i: (i, 0)),
            pl.BlockSpec((BLK, H), lambda i: (i, 0)),
            _W, _B,
            _W, _W, _W2, _W2, _W2, _W2,
            _B, _B, _B, _B, _B, _B,
            _W2, _W2, _W2, _W2, _W2,
            _B, _B, _B, _B, _B,
        ],
        out_specs=pl.BlockSpec((BLK, H), lambda i: (i, 0)),
        out_shape=jax.ShapeDtypeStruct((N_NODES, H), jnp.float32),
    )(gu, emb, we, be, *ws, *bs, *wns, *bns)


def _chain_b(gb, emb, we, be, ws, bs, wns, bns, carry):
    nu_b = NU_ // BLK
    return pl.pallas_call(
        _chain_b_body,
        grid=(NB_ // BLK,),
        in_specs=[
            pl.BlockSpec((BLK, H), lambda i: (i, 0)),
            pl.BlockSpec((BLK, H), lambda i: (i + nu_b, 0)),
            pl.BlockSpec((BLK, H), lambda i: (i + nu_b, 0)),
            _W, _B,
            _W2, _W, _W2, _W2, _W2, _W2,
            _B, _B, _B, _B, _B, _B,
            _W2, _W2, _W2, _W2, _W2,
            _B, _B, _B, _B, _B,
            pl.BlockSpec(memory_space=pl.ANY),
        ],
        out_specs=pl.BlockSpec((BLK, H), lambda i: (i + nu_b, 0)),
        out_shape=jax.ShapeDtypeStruct((N_NODES, H), jnp.float32),
        input_output_aliases={27: 0},
    )(gb, gb, emb, we, be, *ws, *bs, *wns, *bns, carry)


def kernel(node_feats, unary_src, binary_src, params):
    p = params
    bf16 = jnp.bfloat16

    u0, u1 = 15, 11
    b0_, b1_ = 33, 16
    idx_u = _pad_idx(unary_src, u0, u1)
    idx_b = _pad_idx(
        jnp.concatenate([binary_src[:, 0], binary_src[:, 1]]), b0_, b1_
    )
    we = p["We"].astype(bf16)
    be = p["be"].reshape(1, H)
    we, idx_u, idx_b = lax.optimization_barrier((we, idx_u, idx_b))
    gu = _sc_gather(node_feats, idx_u, u0, u1)
    gb = _sc_gather(node_feats, idx_b, b0_, b1_)
    emb = _embed(node_feats, we, be, 2000)

    def wc(n):
        return p["W" + n].astype(bf16)

    def b2d(n):
        return p["b" + n].reshape(1, H)

    wsu = [wc("u%d" % i) for i in range(6)]
    bsu = [b2d("u%d" % i) for i in range(6)]
    wsb = [wc("b%d" % i) for i in range(6)]
    bsb = [b2d("b%d" % i) for i in range(6)]
    wns = [wc("n%d" % i) for i in range(5)]
    bns = [b2d("n%d" % i) for i in range(5)]

    e_u = _chain_u(gu, emb, we, be, wsu, bsu, wns, bns)
    return _chain_b(gb, emb, we, be, wsb, bsb, wns, bns, e_u)


# speedup vs baseline: 1.4427x; 1.0107x over previous
import functools

import jax
import jax.numpy as jnp
from jax import lax
from jax.experimental import pallas as pl
from jax.experimental.pallas import tpu as pltpu
from jax.experimental.pallas import tpu_sc as plsc

H = 128
N_NODES = 100000
NU_ = 50000
NB_ = 50000
BLK = 5000

_NC = 2
_NS = 16
_NW = _NC * _NS
_CH = 128


def _embed_body(x_ref, w_ref, b_ref, o_ref):
    x = x_ref[...].astype(jnp.bfloat16)
    o_ref[...] = jnp.tanh(
        jnp.dot(x, w_ref[...], preferred_element_type=jnp.float32) + b_ref[...]
    ).astype(jnp.bfloat16)


def _embed(x, w, b, blk):
    n = x.shape[0]
    return pl.pallas_call(
        _embed_body,
        grid=(n // blk,),
        in_specs=[
            pl.BlockSpec((blk, H), lambda i: (i, 0)),
            pl.BlockSpec((H, H), lambda i: (0, 0)),
            pl.BlockSpec((1, H), lambda i: (0, 0)),
        ],
        out_specs=pl.BlockSpec((blk, H), lambda i: (i, 0)),
        out_shape=jax.ShapeDtypeStruct((n, H), jnp.bfloat16),
    )(x, w, b)


def _pad_idx(idx_flat, steps0, steps1):
    sp = steps0 + steps1
    total = 16 * sp * _CH
    flat = jnp.concatenate(
        [idx_flat, jnp.zeros((total - idx_flat.shape[0],), jnp.int32)]
    ).reshape(16, sp * _CH)
    maxs = max(steps0, steps1)
    i0 = flat[:, : steps0 * _CH].reshape(16, steps0, _CH)
    i1 = flat[:, steps0 * _CH :].reshape(16, steps1, _CH)
    z0 = jnp.zeros((16, maxs - steps0, _CH), jnp.int32)
    z1 = jnp.zeros((16, maxs - steps1, _CH), jnp.int32)
    i0 = jnp.concatenate([i0, z0], axis=1)
    i1 = jnp.concatenate([i1, z1], axis=1)
    return jnp.stack([i0, i1], axis=1).reshape(_NW, maxs, _CH)


def _sc_gather(table, idx3d, steps0, steps1):
    maxs = idx3d.shape[1]
    total = 16 * (steps0 + steps1) * _CH
    mesh = plsc.VectorSubcoreMesh(core_axis_name="c", subcore_axis_name="s")

    @functools.partial(
        pl.kernel,
        mesh=mesh,
        out_type=jax.ShapeDtypeStruct((total, H), jnp.float32),
        scratch_types=[
            pltpu.VMEM((maxs, _CH), jnp.int32),
            pltpu.VMEM((_CH, H), jnp.float32),
            pltpu.SemaphoreType.DMA,
        ],
    )
    def gather_kernel(table_hbm, idx_hbm, out_hbm, idx_v, rows_v, sem):
        s = lax.axis_index("s")
        c = lax.axis_index("c")
        wid = s * _NC + c
        my_steps = steps0 + c * (steps1 - steps0)
        row0 = s * (steps0 + steps1) + c * steps0
        pltpu.sync_copy(idx_hbm.at[wid], idx_v)

        def body(j, carry):
            pltpu.async_copy(table_hbm.at[idx_v.at[j]], rows_v, sem).wait()
            pltpu.sync_copy(rows_v, out_hbm.at[pl.ds((row0 + j) * _CH, _CH)])
            return carry

        lax.fori_loop(0, my_steps, body, 0)

    return gather_kernel(table, idx3d)


def _dot(a, w):
    return jnp.dot(a, w, preferred_element_type=jnp.float32)


def _t(v):
    return v.astype(jnp.bfloat16)


def _msg_and_node(r0, emb, msg_layers, node_ws, node_bs):
    (w1, b1) = msg_layers[0]
    r = jnp.tanh(_dot(r0, w1[...]) + b1[...])
    for (w_ref, b_ref) in msg_layers[1:]:
        r = jnp.tanh(
            _dot(jnp.concatenate([_t(r), r0], axis=1), w_ref[...]) + b_ref[...]
        )
    r = _t(r)
    e = jnp.tanh(
        _dot(jnp.concatenate([emb, r], axis=1), node_ws[0][...]) + node_bs[0][...]
    )
    for (w_ref, b_ref) in zip(node_ws[1:], node_bs[1:]):
        e = jnp.tanh(
            _dot(jnp.concatenate([_t(e), emb], axis=1), w_ref[...]) + b_ref[...]
        )
    return e


def _chain_u_body(
    xu_ref, emb_ref, we_ref, be_ref,
    w0, w1, w2, w3, w4, w5, b0, b1, b2, b3, b4, b5,
    wn0, wn1, wn2, wn3, wn4, bn0, bn1, bn2, bn3, bn4,
    o_ref,
):
    emb = emb_ref[...]
    m = _t(jnp.tanh(_dot(_t(xu_ref[...]), we_ref[...]) + be_ref[...]))
    r0 = _t(jnp.tanh(_dot(m, w0[...]) + b0[...]))
    o_ref[...] = _msg_and_node(
        r0, emb,
        ((w1, b1), (w2, b2), (w3, b3), (w4, b4), (w5, b5)),
        (wn0, wn1, wn2, wn3, wn4), (bn0, bn1, bn2, bn3, bn4),
    )


def _chain_b_body(
    xb1_ref, xb2_ref, emb_ref, we_ref, be_ref,
    w0, w1, w2, w3, w4, w5, b0, b1, b2, b3, b4, b5,
    wn0, wn1, wn2, wn3, wn4, bn0, bn1, bn2, bn3, bn4,
    carry_ref, o_ref,
):
    emb = emb_ref[...]
    m1 = _t(jnp.tanh(_dot(_t(xb1_ref[...]), we_ref[...]) + be_ref[...]))
    m2 = _t(jnp.tanh(_dot(_t(xb2_ref[...]), we_ref[...]) + be_ref[...]))
    s0 = _t(
        jnp.tanh(_dot(jnp.concatenate([m1, m2], axis=1), w0[...]) + b0[...])
    )
    o_ref[...] = _msg_and_node(
        s0, emb,
        ((w1, b1), (w2, b2), (w3, b3), (w4, b4), (w5, b5)),
        (wn0, wn1, wn2, wn3, wn4), (bn0, bn1, bn2, bn3, bn4),
    )


_W = pl.BlockSpec((H, H), lambda i: (0, 0))
_W2 = pl.BlockSpec((2 * H, H), lambda i: (0, 0))
_B = pl.BlockSpec((1, H), lambda i: (0, 0))


def _chain_u(gu, emb, we, be, ws, bs, wns, bns):
    return pl.pallas_call(
        _chain_u_body,
        grid=(NU_ // BLK,),
        in_specs=[
            pl.BlockSpec((BLK, H), lambda i: (i, 0)),
            pl.BlockSpec((BLK, H), lambda i: (i, 0)),
            _W, _B,
            _W, _W, _W2, _W2, _W2, _W2,
            _B, _B, _B, _B, _B, _B,
            _W2, _W2, _W2, _W2, _W2,
            _B, _B, _B, _B, _B,
        ],
        out_specs=pl.BlockSpec((BLK, H), lambda i: (i, 0)),
        out_shape=jax.ShapeDtypeStruct((N_NODES, H), jnp.float32),
    )(gu, emb, we, be, *ws, *bs, *wns, *bns)


def _chain_b(gb, emb, we, be, ws, bs, wns, bns, carry):
    nu_b = NU_ // BLK
    return pl.pallas_call(
        _chain_b_body,
        grid=(NB_ // BLK,),
        in_specs=[
            pl.BlockSpec((BLK, H), lambda i: (i, 0)),
            pl.BlockSpec((BLK, H), lambda i: (i + nu_b, 0)),
            pl.BlockSpec((BLK, H), lambda i: (i + nu_b, 0)),
            _W, _B,
            _W2, _W, _W2, _W2, _W2, _W2,
            _B, _B, _B, _B, _B, _B,
            _W2, _W2, _W2, _W2, _W2,
            _B, _B, _B, _B, _B,
            pl.BlockSpec(memory_space=pl.ANY),
        ],
        out_specs=pl.BlockSpec((BLK, H), lambda i: (i + nu_b, 0)),
        out_shape=jax.ShapeDtypeStruct((N_NODES, H), jnp.float32),
        input_output_aliases={27: 0},
    )(gb, gb, emb, we, be, *ws, *bs, *wns, *bns, carry)


def kernel(node_feats, unary_src, binary_src, params):
    p = params
    bf16 = jnp.bfloat16

    u0, u1 = 15, 11
    b0_, b1_ = 33, 16
    idx_u = _pad_idx(unary_src, u0, u1)
    idx_b = _pad_idx(
        jnp.concatenate([binary_src[:, 0], binary_src[:, 1]]), b0_, b1_
    )
    we = p["We"].astype(bf16)
    be = p["be"].reshape(1, H)
    we, idx_u, idx_b = lax.optimization_barrier((we, idx_u, idx_b))
    gu = _sc_gather(node_feats, idx_u, u0, u1)
    gb = _sc_gather(node_feats, idx_b, b0_, b1_)
    emb = _embed(node_feats, we, be, 2000)

    def wc(n):
        return p["W" + n].astype(bf16)

    def b2d(n):
        return p["b" + n].reshape(1, H)

    wsu = [wc("u%d" % i) for i in range(6)]
    bsu = [b2d("u%d" % i) for i in range(6)]
    wsb = [wc("b%d" % i) for i in range(6)]
    bsb = [b2d("b%d" % i) for i in range(6)]
    wns = [wc("n%d" % i) for i in range(5)]
    bns = [b2d("n%d" % i) for i in range(5)]

    e_u = _chain_u(gu, emb, we, be, wsu, bsu, wns, bns)
    return _chain_b(gb, emb, we, be, wsb, bsb, wns, bns, e_u)
